# Initial kernel scaffold; baseline (speedup 1.0000x reference)
import jax
import jax.numpy as jnp
from jax.experimental import pallas as pl


def kernel(z, codebook):
    raise NotImplementedError("write your pallas kernel here")


# speedup vs baseline: 9.1534x; 9.1534x over previous
import functools

import jax
import jax.numpy as jnp
from jax import lax
from jax.experimental import pallas as pl
from jax.experimental.pallas import tpu as pltpu
from jax.experimental.pallas import tpu_sc as plsc

N_TOKENS = 16384
N_CODES = 8192
DIM = 32
TOKEN_BLOCK = 256


@functools.cache
def _make_sc_gather():
    nc, ns = 2, 16
    nw = nc * ns
    b_per_w = N_TOKENS // nw
    mesh = plsc.VectorSubcoreMesh(core_axis_name="c", subcore_axis_name="s")

    chunk = 128
    n_chunks = b_per_w // chunk

    @functools.partial(
        pl.kernel,
        out_type=jax.ShapeDtypeStruct((N_TOKENS, DIM), jnp.float32),
        mesh=mesh,
        scratch_types=[
            pltpu.VMEM((n_chunks, chunk), jnp.int32),
            pltpu.VMEM((b_per_w, DIM), jnp.float32),
            pltpu.SemaphoreType.DMA,
        ],
        compiler_params=pltpu.CompilerParams(use_tc_tiling_on_sc=False),
    )
    def gather(table_hbm, idx_hbm, out_hbm, idx_v, rows_v, sem):
        wid = lax.axis_index("s") * nc + lax.axis_index("c")
        base = wid * n_chunks
        pltpu.sync_copy(idx_hbm.at[pl.ds(base, n_chunks)], idx_v)
        copies = [
            pltpu.async_copy(
                table_hbm.at[idx_v.at[j]],
                rows_v.at[pl.ds(j * chunk, chunk)], sem)
            for j in range(n_chunks)
        ]
        for c in copies:
            c.wait()
        pltpu.sync_copy(rows_v, out_hbm.at[pl.ds(base * chunk, b_per_w)])

    return gather


def kernel(z, codebook):
    b, d, h, w = z.shape
    zf = jnp.transpose(z, (0, 2, 3, 1)).reshape(b * h * w, d)
    distances = (jnp.sum(zf ** 2, axis=1, keepdims=True)
                 + jnp.sum(codebook ** 2, axis=1)
                 - 2.0 * jnp.matmul(zf, codebook.T))
    idx = jnp.argmin(distances, axis=1).astype(jnp.int32)
    rows = _make_sc_gather()(codebook, idx.reshape(N_TOKENS // 128, 128))
    return rows.reshape(b, h, w, d).transpose(0, 3, 1, 2)


# speedup vs baseline: 11.2951x; 1.2340x over previous
import functools

import jax
import jax.numpy as jnp
from jax import lax
from jax.experimental import pallas as pl
from jax.experimental.pallas import tpu as pltpu
from jax.experimental.pallas import tpu_sc as plsc

N_TOKENS = 16384
N_CODES = 8192
DIM = 32
TOKEN_BLOCK = 256
CHUNK = 4096


def _argmin_body(zfb_ref, zsq_ref, cb_ref, csq_ref, idx_ref):
    mm = lax.dot_general(
        zfb_ref[...], cb_ref[...],
        dimension_numbers=(((1,), (1,)), ((), ())),
        preferred_element_type=jnp.float32,
    )
    dist = (zsq_ref[...] + csq_ref[...]) - 2.0 * mm
    acc_i = None
    carry = None
    for k in range(N_CODES // CHUNK):
        blk = dist[:, k * CHUNK:(k + 1) * CHUNK]
        m = jnp.min(blk, axis=1, keepdims=True)
        iota = lax.broadcasted_iota(jnp.int32, blk.shape, 1) + k * CHUNK
        ik = jnp.min(jnp.where(blk == m, iota, N_CODES), axis=1, keepdims=True)
        stored = m.astype(jnp.bfloat16).astype(jnp.float32)
        if k == 0:
            acc_i, carry = ik, stored
        else:
            take = m < carry
            acc_i = jnp.where(take, ik, acc_i)
            carry = jnp.where(take, stored, carry)
    idx_ref[...] = acc_i


def _compute_indices(zfb, zsq, cb, csq):
    return pl.pallas_call(
        _argmin_body,
        grid=(N_TOKENS // TOKEN_BLOCK,),
        in_specs=[
            pl.BlockSpec((TOKEN_BLOCK, DIM), lambda i: (i, 0)),
            pl.BlockSpec((TOKEN_BLOCK, 1), lambda i: (i, 0)),
            pl.BlockSpec((N_CODES, DIM), lambda i: (0, 0)),
            pl.BlockSpec((1, N_CODES), lambda i: (0, 0)),
        ],
        out_specs=pl.BlockSpec((TOKEN_BLOCK, 1), lambda i: (i, 0)),
        out_shape=jax.ShapeDtypeStruct((N_TOKENS, 1), jnp.int32),
    )(zfb, zsq, cb, csq)


@functools.cache
def _make_sc_gather():
    nc, ns = 2, 16
    nw = nc * ns
    b_per_w = N_TOKENS // nw
    mesh = plsc.VectorSubcoreMesh(core_axis_name="c", subcore_axis_name="s")

    chunk = 128
    n_chunks = b_per_w // chunk

    @functools.partial(
        pl.kernel,
        out_type=jax.ShapeDtypeStruct((N_TOKENS, DIM), jnp.float32),
        mesh=mesh,
        scratch_types=[
            pltpu.VMEM((n_chunks, chunk), jnp.int32),
            pltpu.VMEM((b_per_w, DIM), jnp.float32),
            pltpu.SemaphoreType.DMA,
        ],
        compiler_params=pltpu.CompilerParams(use_tc_tiling_on_sc=False),
    )
    def gather(table_hbm, idx_hbm, out_hbm, idx_v, rows_v, sem):
        wid = lax.axis_index("s") * nc + lax.axis_index("c")
        base = wid * n_chunks
        pltpu.sync_copy(idx_hbm.at[pl.ds(base, n_chunks)], idx_v)
        copies = [
            pltpu.async_copy(
                table_hbm.at[idx_v.at[j]],
                rows_v.at[pl.ds(j * chunk, chunk)], sem)
            for j in range(n_chunks)
        ]
        for c in copies:
            c.wait()
        pltpu.sync_copy(rows_v, out_hbm.at[pl.ds(base * chunk, b_per_w)])

    return gather


def kernel(z, codebook):
    b, d, h, w = z.shape
    zf = jnp.transpose(z, (0, 2, 3, 1)).reshape(b * h * w, d)
    zsq = jnp.sum(zf ** 2, axis=1, keepdims=True)
    csq = jnp.sum(codebook ** 2, axis=1).reshape(1, N_CODES)
    zfb = zf.astype(jnp.bfloat16)
    idx = _compute_indices(zfb, zsq, codebook, csq)
    rows = _make_sc_gather()(codebook, idx.reshape(N_TOKENS // 128, 128))
    return rows.reshape(b, h, w, d).transpose(0, 3, 1, 2)
